# Initial kernel scaffold; baseline (speedup 1.0000x reference)
import jax
import jax.numpy as jnp
from jax.experimental import pallas as pl


def kernel(code_list, tables):
    raise NotImplementedError("write your pallas kernel here")


# speedup vs baseline: 25.2941x; 25.2941x over previous
import functools

import jax
import jax.numpy as jnp
from jax import lax
from jax.experimental import pallas as pl
from jax.experimental.pallas import tpu as pltpu
from jax.experimental.pallas import tpu_sc as plsc

D_SIZE = 8
MC_SIZE = 8192
PQ_DIM = 64
BATCH = 16384

NC = 2
NS = 16
NW = NC * NS
B_PER_W = BATCH // NW
CHUNK = 128
NCHUNK = B_PER_W // CHUNK
ROWS = D_SIZE * NCHUNK
LANES = 16


def _pq_body(codes_hbm, tables_hbm, out_hbm, idx_v, acc_v, sem_init, sem_add):
    wid = lax.axis_index("s") * NC + lax.axis_index("c")
    base = wid * B_PER_W

    pltpu.sync_copy(codes_hbm.at[wid], idx_v)

    init = [
        pltpu.async_copy(
            tables_hbm.at[idx_v.at[j]],
            acc_v.at[pl.ds(j * CHUNK, CHUNK)],
            sem_init,
        )
        for j in range(NCHUNK)
    ]
    for d in init:
        d.wait()

    adds = [
        pltpu.async_copy(
            tables_hbm.at[idx_v.at[i * NCHUNK + j]],
            acc_v.at[pl.ds(j * CHUNK, CHUNK)],
            sem_add,
            add=True,
        )
        for i in range(1, D_SIZE)
        for j in range(NCHUNK)
    ]
    for d in adds:
        d.wait()

    def scale_row(r, carry):
        for c in range(PQ_DIM // LANES):
            sl = pl.ds(c * LANES, LANES)
            acc_v[r, sl] = acc_v[r, sl] * 0.125
        return carry

    lax.fori_loop(0, B_PER_W, scale_row, 0, unroll=4)

    pltpu.sync_copy(acc_v, out_hbm.at[pl.ds(base, B_PER_W)])


_pq_call = pl.kernel(
    _pq_body,
    out_type=jax.ShapeDtypeStruct((BATCH, PQ_DIM), jnp.float32),
    mesh=plsc.VectorSubcoreMesh(core_axis_name="c", subcore_axis_name="s"),
    scratch_types=[
        pltpu.VMEM((ROWS, CHUNK), jnp.int32),
        pltpu.VMEM((B_PER_W, PQ_DIM), jnp.float32),
        pltpu.SemaphoreType.DMA,
        pltpu.SemaphoreType.DMA,
    ],
    compiler_params=pltpu.CompilerParams(use_tc_tiling_on_sc=False),
)


@jax.jit
def kernel(code_list, tables):
    offs = (jnp.arange(D_SIZE, dtype=jnp.int32) * MC_SIZE)[:, None]
    flat_codes = code_list.astype(jnp.int32) + offs
    codes3 = (
        flat_codes.reshape(D_SIZE, NW, NCHUNK, CHUNK)
        .transpose(1, 0, 2, 3)
        .reshape(NW, ROWS, CHUNK)
    )
    tables_flat = tables.reshape(D_SIZE * MC_SIZE, PQ_DIM)
    return _pq_call(codes3, tables_flat)
